# baseline (speedup 1.0000x reference)
import functools

import jax
import jax.numpy as jnp
from jax import lax
from jax.experimental import pallas as pl
from jax.experimental.pallas import tpu as pltpu
from jax.experimental.pallas import tpu_sc as plsc

NUM_EXPERTS = 8
TOP_K = 2
D_MODEL = 2048
N_TOKENS = 8192
N_ASSIGN = N_TOKENS * TOP_K

BT = 512
BROW = 256
L_PAD = N_ASSIGN + NUM_EXPERTS * BROW
G_BLOCKS = L_PAD // BROW

SC_CORES = 2
SC_SUBCORES = 16
SC_WORKERS = SC_CORES * SC_SUBCORES


def _gate_body(x_ref, wg_ref, w_out_ref, sel_out_ref, rank_out_ref,
               cnt_out_ref, cnt_ref):
    t = pl.program_id(0)

    @pl.when(t == 0)
    def _():
        cnt_ref[...] = jnp.zeros((1, NUM_EXPERTS), jnp.float32)

    x = x_ref[...]
    logits = jax.lax.dot_general(
        x, wg_ref[...], (((1,), (1,)), ((), ())),
        preferred_element_type=jnp.float32)
    neg_inf = jnp.float32(-jnp.inf)
    m1 = jnp.full((BT,), neg_inf)
    a1 = jnp.zeros((BT,), jnp.float32)
    for j in range(NUM_EXPERTS):
        lj = logits[:, j]
        better = lj > m1
        m1 = jnp.where(better, lj, m1)
        a1 = jnp.where(better, jnp.float32(j), a1)
    m2 = jnp.full((BT,), neg_inf)
    a2 = jnp.zeros((BT,), jnp.float32)
    for j in range(NUM_EXPERTS):
        lj = logits[:, j]
        valid = jnp.float32(j) != a1
        better = (lj > m2) & valid
        m2 = jnp.where(better, lj, m2)
        a2 = jnp.where(better, jnp.float32(j), a2)
    e2 = jnp.exp(m2 - m1)
    w1 = 1.0 / (1.0 + e2)
    w2 = 1.0 - w1
    w_out_ref[...] = jnp.stack([w1, w2], axis=1)
    sel_out_ref[...] = jnp.stack([a1, a2], axis=1).astype(jnp.int32)

    iota_e = jax.lax.broadcasted_iota(
        jnp.int32, (BT, NUM_EXPERTS), 1).astype(jnp.float32)
    oh1 = (a1[:, None] == iota_e).astype(jnp.float32)
    oh2 = (a2[:, None] == iota_e).astype(jnp.float32)
    oha = jnp.stack([oh1, oh2], axis=1).reshape(2 * BT, NUM_EXPERTS)
    ri = jax.lax.broadcasted_iota(jnp.int32, (2 * BT, 2 * BT), 0)
    ci = jax.lax.broadcasted_iota(jnp.int32, (2 * BT, 2 * BT), 1)
    tri = (ci < ri).astype(jnp.float32)
    ranks_local = jax.lax.dot_general(
        tri, oha, (((1,), (0,)), ((), ())),
        preferred_element_type=jnp.float32)
    rank_a = jnp.sum((ranks_local + cnt_ref[...]) * oha, axis=1)
    rank_out_ref[...] = rank_a.reshape(BT, TOP_K).astype(jnp.int32)
    cnt_ref[...] += jnp.sum(oha, axis=0, keepdims=True)
    cnt_out_ref[...] = cnt_ref[...]


def _gate(x, Wg):
    return pl.pallas_call(
        _gate_body,
        grid=(N_TOKENS // BT,),
        in_specs=[
            pl.BlockSpec((BT, D_MODEL), lambda t: (t, 0)),
            pl.BlockSpec((NUM_EXPERTS, D_MODEL), lambda t: (0, 0)),
        ],
        out_specs=[
            pl.BlockSpec((BT, TOP_K), lambda t: (t, 0)),
            pl.BlockSpec((BT, TOP_K), lambda t: (t, 0)),
            pl.BlockSpec((BT, TOP_K), lambda t: (t, 0)),
            pl.BlockSpec((1, NUM_EXPERTS), lambda t: (0, 0)),
        ],
        out_shape=[
            jax.ShapeDtypeStruct((N_TOKENS, TOP_K), jnp.float32),
            jax.ShapeDtypeStruct((N_TOKENS, TOP_K), jnp.int32),
            jax.ShapeDtypeStruct((N_TOKENS, TOP_K), jnp.int32),
            jax.ShapeDtypeStruct((1, NUM_EXPERTS), jnp.float32),
        ],
        scratch_shapes=[pltpu.VMEM((1, NUM_EXPERTS), jnp.float32)],
        compiler_params=pltpu.CompilerParams(
            dimension_semantics=("arbitrary",)),
    )(x, Wg)


def _routing_metadata(w2, sel2, rank2, cnt):
    e_flat = sel2.reshape(-1)
    w_flat = w2.reshape(-1)
    rank = rank2.reshape(-1)
    t_flat = jnp.arange(N_ASSIGN, dtype=jnp.int32) // TOP_K
    counts = cnt[0].astype(jnp.int32)
    caps = ((counts + BROW - 1) // BROW) * BROW
    P = jnp.concatenate([jnp.zeros((1,), jnp.int32),
                         jnp.cumsum(caps).astype(jnp.int32)])
    pos = P[e_flat] + rank
    tok_pad = jnp.zeros((L_PAD,), jnp.int32).at[pos].set(t_flat)
    wgt_pad = jnp.zeros((L_PAD,), jnp.float32).at[pos].set(w_flat)
    row0 = jnp.arange(G_BLOCKS, dtype=jnp.int32) * BROW
    block_expert = jnp.clip(
        jnp.searchsorted(P, row0, side="right").astype(jnp.int32) - 1,
        0, NUM_EXPERTS - 1)
    return tok_pad, wgt_pad, block_expert, pos


DISP_CH = 16
DISP_NBUF = 3
N_HALF = L_PAD // 2


def _make_dispatch_body(row0, nrows):
    rw = nrows // SC_WORKERS
    nch = rw // DISP_CH

    def body(x_hbm, tok_hbm, xs_hbm, idx_v, *rest):
        bufs = rest[:DISP_NBUF]
        gsems = rest[DISP_NBUF:2 * DISP_NBUF]
        wsems = rest[2 * DISP_NBUF:3 * DISP_NBUF]
        wid = lax.axis_index("s") * SC_CORES + lax.axis_index("c")
        base = wid * rw
        pltpu.sync_copy(tok_hbm.at[pl.ds(row0 + base, rw)], idx_v)

        def gather_desc(c, b):
            return pltpu.make_async_copy(
                x_hbm.at[idx_v.at[pl.ds(c * DISP_CH, DISP_CH)]],
                bufs[b], gsems[b])

        def write_desc(c, b):
            return pltpu.make_async_copy(
                bufs[b], xs_hbm.at[pl.ds(base + c * DISP_CH, DISP_CH)],
                wsems[b])

        for r in range(DISP_NBUF):
            gather_desc(r, r).start()

        @pl.loop(0, nch, step=DISP_NBUF)
        def _(q):
            for r in range(DISP_NBUF):
                gather_desc(q + r, r).wait()
                write_desc(q + r, r).start()
            for r in range(DISP_NBUF):
                @pl.when(q + r + DISP_NBUF < nch)
                def _():
                    write_desc(q + r, r).wait()
                    gather_desc(q + r + DISP_NBUF, r).start()

        for r in range(DISP_NBUF):
            write_desc(nch - DISP_NBUF + r, r).wait()

    return body


def _sc_dispatch(x, tok_pad, row0, nrows):
    mesh = plsc.VectorSubcoreMesh(core_axis_name="c", subcore_axis_name="s")
    rw = nrows // SC_WORKERS
    f = pl.kernel(
        _make_dispatch_body(row0, nrows),
        jax.ShapeDtypeStruct((nrows, D_MODEL), jnp.float32),
        mesh=mesh,
        scratch_types=(
            [pltpu.VMEM((rw,), jnp.int32)]
            + [pltpu.VMEM((DISP_CH, D_MODEL), jnp.float32)] * DISP_NBUF
            + [pltpu.SemaphoreType.DMA] * (2 * DISP_NBUF)
        ),
    )
    return f(x, tok_pad)


def _gmm_body(be_ref, xs_ref, w_ref, b_ref, wgt_ref, o_ref):
    del be_ref
    xs = xs_ref[...]
    y = jax.lax.dot_general(
        xs, w_ref[0], (((1,), (1,)), ((), ())),
        preferred_element_type=jnp.float32) + b_ref[0]
    o_ref[...] = y * wgt_ref[0, 0][:, None]


def _gmm_body2(be_ref, xs_ref, w_ref, b_ref, wgt_ref, prev_ref, o_ref):
    del prev_ref
    _gmm_body(be_ref, xs_ref, w_ref, b_ref, wgt_ref, o_ref)


def _grouped_matmul_half(xs_half, W, b, wgt_half, block_expert, g0, ys_prev):
    hg = G_BLOCKS // 2
    in_specs = [
        pl.BlockSpec((BROW, D_MODEL), lambda g, be: (g, 0)),
        pl.BlockSpec((1, D_MODEL, D_MODEL), lambda g, be: (be[g0 + g], 0, 0)),
        pl.BlockSpec((1, 1, D_MODEL), lambda g, be: (be[g0 + g], 0, 0)),
        pl.BlockSpec((1, 1, BROW), lambda g, be: (g, 0, 0)),
    ]
    args = [block_expert, xs_half, W, b.reshape(NUM_EXPERTS, 1, D_MODEL),
            wgt_half.reshape(hg, 1, BROW)]
    body = _gmm_body
    aliases = {}
    if ys_prev is not None:
        in_specs = in_specs + [pl.BlockSpec(memory_space=pl.ANY)]
        args = args + [ys_prev]
        body = _gmm_body2
        aliases = {5: 0}
    return pl.pallas_call(
        body,
        grid_spec=pltpu.PrefetchScalarGridSpec(
            num_scalar_prefetch=1,
            grid=(hg,),
            in_specs=in_specs,
            out_specs=pl.BlockSpec((BROW, D_MODEL), lambda g, be: (g0 + g, 0)),
        ),
        out_shape=jax.ShapeDtypeStruct((L_PAD, D_MODEL), jnp.float32),
        input_output_aliases=aliases,
        compiler_params=pltpu.CompilerParams(
            dimension_semantics=("arbitrary",)),
    )(*args)


COMB_CT = 8
COMB_TW = N_TOKENS // SC_WORKERS


def _combine_body(ys_hbm, pos_hbm, out_hbm, pidx_v, g0, g1, ob0, ob1,
                  gsem0, gsem1, wsem0, wsem1):
    nch = COMB_TW // COMB_CT
    wid = lax.axis_index("s") * SC_CORES + lax.axis_index("c")
    base_t = wid * COMB_TW
    pltpu.sync_copy(pos_hbm.at[pl.ds(TOP_K * base_t, TOP_K * COMB_TW)], pidx_v)
    gbufs = (g0, g1)
    obufs = (ob0, ob1)
    gsems = (gsem0, gsem1)
    wsems = (wsem0, wsem1)

    def gather_desc(c, b):
        return pltpu.make_async_copy(
            ys_hbm.at[pidx_v.at[pl.ds(c * TOP_K * COMB_CT, TOP_K * COMB_CT)]],
            gbufs[b], gsems[b])

    def write_desc(c, b):
        return pltpu.make_async_copy(
            obufs[b], out_hbm.at[pl.ds(base_t + c * COMB_CT, COMB_CT)], wsems[b])

    def compute(b):
        gb = gbufs[b]
        obuf = obufs[b]
        for t in range(COMB_CT):
            @plsc.parallel_loop(0, D_MODEL // 16, unroll=8)
            def _(j):
                sl = pl.ds(j * 16, 16)
                obuf[t, sl] = gb[2 * t, sl] + gb[2 * t + 1, sl]

    gather_desc(0, 0).start()

    @pl.loop(0, nch, step=2)
    def _(q):
        gather_desc(q, 0).wait()
        gather_desc(q + 1, 1).start()

        @pl.when(q >= 2)
        def _():
            write_desc(q - 2, 0).wait()
        compute(0)
        write_desc(q, 0).start()

        @pl.when(q + 2 < nch)
        def _():
            gather_desc(q + 2, 0).start()
        gather_desc(q + 1, 1).wait()

        @pl.when(q >= 1)
        def _():
            write_desc(q - 1, 1).wait()
        compute(1)
        write_desc(q + 1, 1).start()

    write_desc(nch - 2, 0).wait()
    write_desc(nch - 1, 1).wait()


def _sc_combine(ys, pos):
    mesh = plsc.VectorSubcoreMesh(core_axis_name="c", subcore_axis_name="s")
    f = pl.kernel(
        _combine_body,
        jax.ShapeDtypeStruct((N_TOKENS, D_MODEL), jnp.float32),
        mesh=mesh,
        scratch_types=[
            pltpu.VMEM((TOP_K * COMB_TW,), jnp.int32),
            pltpu.VMEM((TOP_K * COMB_CT, D_MODEL), jnp.float32),
            pltpu.VMEM((TOP_K * COMB_CT, D_MODEL), jnp.float32),
            pltpu.VMEM((COMB_CT, D_MODEL), jnp.float32),
            pltpu.VMEM((COMB_CT, D_MODEL), jnp.float32),
            pltpu.SemaphoreType.DMA,
            pltpu.SemaphoreType.DMA,
            pltpu.SemaphoreType.DMA,
            pltpu.SemaphoreType.DMA,
        ],
    )
    return f(ys, pos)


@jax.jit
def kernel(x, Wg, W, b):
    w2, sel2, rank2, cnt = _gate(x, Wg)
    tok_pad, wgt_pad, block_expert, pos = _routing_metadata(w2, sel2, rank2, cnt)
    xs1 = _sc_dispatch(x, tok_pad, 0, N_HALF)
    xs2 = _sc_dispatch(x, tok_pad, N_HALF, N_HALF)
    ys1 = _grouped_matmul_half(xs1, W, b, wgt_pad[:N_HALF], block_expert,
                               0, None)
    ys = _grouped_matmul_half(xs2, W, b, wgt_pad[N_HALF:], block_expert,
                              G_BLOCKS // 2, ys1)
    return _sc_combine(ys, pos)


# speedup vs baseline: 1.0488x; 1.0488x over previous
import functools

import jax
import jax.numpy as jnp
from jax import lax
from jax.experimental import pallas as pl
from jax.experimental.pallas import tpu as pltpu
from jax.experimental.pallas import tpu_sc as plsc

NUM_EXPERTS = 8
TOP_K = 2
D_MODEL = 2048
N_TOKENS = 8192
N_ASSIGN = N_TOKENS * TOP_K

BT = 512
BROW = 256
L_PAD = N_ASSIGN + NUM_EXPERTS * BROW
G_BLOCKS = L_PAD // BROW

SC_CORES = 2
SC_SUBCORES = 16
SC_WORKERS = SC_CORES * SC_SUBCORES


def _gate_body(x_ref, wg_ref, w_out_ref, sel_out_ref, rank_out_ref,
               cnt_out_ref, cnt_ref, tri_ref):
    t = pl.program_id(0)

    @pl.when(t == 0)
    def _():
        cnt_ref[...] = jnp.zeros((1, NUM_EXPERTS), jnp.float32)
        ri = jax.lax.broadcasted_iota(jnp.int32, (2 * BT, 2 * BT), 0)
        ci = jax.lax.broadcasted_iota(jnp.int32, (2 * BT, 2 * BT), 1)
        tri_ref[...] = (ci < ri).astype(jnp.float32)

    x = x_ref[...]
    logits = jax.lax.dot_general(
        x, wg_ref[...], (((1,), (1,)), ((), ())),
        preferred_element_type=jnp.float32)
    neg_inf = jnp.float32(-jnp.inf)
    m1 = jnp.full((BT,), neg_inf)
    a1 = jnp.zeros((BT,), jnp.float32)
    for j in range(NUM_EXPERTS):
        lj = logits[:, j]
        better = lj > m1
        m1 = jnp.where(better, lj, m1)
        a1 = jnp.where(better, jnp.float32(j), a1)
    m2 = jnp.full((BT,), neg_inf)
    a2 = jnp.zeros((BT,), jnp.float32)
    for j in range(NUM_EXPERTS):
        lj = logits[:, j]
        valid = jnp.float32(j) != a1
        better = (lj > m2) & valid
        m2 = jnp.where(better, lj, m2)
        a2 = jnp.where(better, jnp.float32(j), a2)
    e2 = jnp.exp(m2 - m1)
    w1 = 1.0 / (1.0 + e2)
    w2 = 1.0 - w1
    w_out_ref[...] = jnp.stack([w1, w2], axis=1)
    sel_out_ref[...] = jnp.stack([a1, a2], axis=1).astype(jnp.int32)

    iota_e = jax.lax.broadcasted_iota(
        jnp.int32, (BT, NUM_EXPERTS), 1).astype(jnp.float32)
    oh1 = (a1[:, None] == iota_e).astype(jnp.float32)
    oh2 = (a2[:, None] == iota_e).astype(jnp.float32)
    oha = jnp.concatenate([oh1, oh2], axis=0)
    ranks_local = jax.lax.dot_general(
        tri_ref[...], oha, (((1,), (0,)), ((), ())),
        preferred_element_type=jnp.float32)
    rank_a = jnp.sum((ranks_local + cnt_ref[...]) * oha, axis=1)
    rank_out_ref[...] = jnp.stack(
        [rank_a[:BT], rank_a[BT:]], axis=1).astype(jnp.int32)
    cnt_ref[...] += jnp.sum(oha, axis=0, keepdims=True)
    cnt_out_ref[...] = cnt_ref[...]


def _gate(x, Wg):
    return pl.pallas_call(
        _gate_body,
        grid=(N_TOKENS // BT,),
        in_specs=[
            pl.BlockSpec((BT, D_MODEL), lambda t: (t, 0)),
            pl.BlockSpec((NUM_EXPERTS, D_MODEL), lambda t: (0, 0)),
        ],
        out_specs=[
            pl.BlockSpec((BT, TOP_K), lambda t: (t, 0)),
            pl.BlockSpec((BT, TOP_K), lambda t: (t, 0)),
            pl.BlockSpec((BT, TOP_K), lambda t: (t, 0)),
            pl.BlockSpec((1, NUM_EXPERTS), lambda t: (0, 0)),
        ],
        out_shape=[
            jax.ShapeDtypeStruct((N_TOKENS, TOP_K), jnp.float32),
            jax.ShapeDtypeStruct((N_TOKENS, TOP_K), jnp.int32),
            jax.ShapeDtypeStruct((N_TOKENS, TOP_K), jnp.int32),
            jax.ShapeDtypeStruct((1, NUM_EXPERTS), jnp.float32),
        ],
        scratch_shapes=[pltpu.VMEM((1, NUM_EXPERTS), jnp.float32),
                        pltpu.VMEM((2 * BT, 2 * BT), jnp.float32)],
        compiler_params=pltpu.CompilerParams(
            dimension_semantics=("arbitrary",)),
    )(x, Wg)


def _routing_metadata(w2, sel2, rank2, cnt):
    e_flat = sel2.reshape(-1)
    w_flat = w2.reshape(-1)
    rank = rank2.reshape(-1)
    t_flat = jnp.arange(N_ASSIGN, dtype=jnp.int32) // TOP_K
    counts = cnt[0].astype(jnp.int32)
    caps = ((counts + BROW - 1) // BROW) * BROW
    P = jnp.concatenate([jnp.zeros((1,), jnp.int32),
                         jnp.cumsum(caps).astype(jnp.int32)])
    pos = P[e_flat] + rank
    tok_pad = jnp.zeros((L_PAD,), jnp.int32).at[pos].set(t_flat)
    wgt_pad = jnp.zeros((L_PAD,), jnp.float32).at[pos].set(w_flat)
    row0 = jnp.arange(G_BLOCKS, dtype=jnp.int32) * BROW
    block_expert = jnp.clip(
        jnp.searchsorted(P, row0, side="right").astype(jnp.int32) - 1,
        0, NUM_EXPERTS - 1)
    return tok_pad, wgt_pad, block_expert, pos


DISP_CH = 16
DISP_NBUF = 3
N_HALF = L_PAD // 2


def _make_dispatch_body(row0, nrows):
    rw = nrows // SC_WORKERS
    nch = rw // DISP_CH

    def body(x_hbm, tok_hbm, xs_hbm, idx_v, *rest):
        bufs = rest[:DISP_NBUF]
        gsems = rest[DISP_NBUF:2 * DISP_NBUF]
        wsems = rest[2 * DISP_NBUF:3 * DISP_NBUF]
        wid = lax.axis_index("s") * SC_CORES + lax.axis_index("c")
        base = wid * rw
        pltpu.sync_copy(tok_hbm.at[pl.ds(row0 + base, rw)], idx_v)

        def gather_desc(c, b):
            return pltpu.make_async_copy(
                x_hbm.at[idx_v.at[pl.ds(c * DISP_CH, DISP_CH)]],
                bufs[b], gsems[b])

        def write_desc(c, b):
            return pltpu.make_async_copy(
                bufs[b], xs_hbm.at[pl.ds(base + c * DISP_CH, DISP_CH)],
                wsems[b])

        for r in range(DISP_NBUF):
            gather_desc(r, r).start()

        @pl.loop(0, nch, step=DISP_NBUF)
        def _(q):
            for r in range(DISP_NBUF):
                gather_desc(q + r, r).wait()
                write_desc(q + r, r).start()
            for r in range(DISP_NBUF):
                @pl.when(q + r + DISP_NBUF < nch)
                def _():
                    write_desc(q + r, r).wait()
                    gather_desc(q + r + DISP_NBUF, r).start()

        for r in range(DISP_NBUF):
            write_desc(nch - DISP_NBUF + r, r).wait()

    return body


def _sc_dispatch(x, tok_pad, row0, nrows):
    mesh = plsc.VectorSubcoreMesh(core_axis_name="c", subcore_axis_name="s")
    rw = nrows // SC_WORKERS
    f = pl.kernel(
        _make_dispatch_body(row0, nrows),
        jax.ShapeDtypeStruct((nrows, D_MODEL), jnp.float32),
        mesh=mesh,
        scratch_types=(
            [pltpu.VMEM((rw,), jnp.int32)]
            + [pltpu.VMEM((DISP_CH, D_MODEL), jnp.float32)] * DISP_NBUF
            + [pltpu.SemaphoreType.DMA] * (2 * DISP_NBUF)
        ),
    )
    return f(x, tok_pad)


def _gmm_body(be_ref, xs_ref, w_ref, b_ref, wgt_ref, o_ref):
    del be_ref
    xs = xs_ref[...]
    y = jax.lax.dot_general(
        xs, w_ref[0], (((1,), (1,)), ((), ())),
        preferred_element_type=jnp.float32) + b_ref[0]
    o_ref[...] = y * wgt_ref[0, 0][:, None]


def _gmm_body2(be_ref, xs_ref, w_ref, b_ref, wgt_ref, prev_ref, o_ref):
    del prev_ref
    _gmm_body(be_ref, xs_ref, w_ref, b_ref, wgt_ref, o_ref)


def _grouped_matmul_half(xs_half, W, b, wgt_half, block_expert, g0, ys_prev):
    hg = G_BLOCKS // 2
    in_specs = [
        pl.BlockSpec((BROW, D_MODEL), lambda g, be: (g, 0)),
        pl.BlockSpec((1, D_MODEL, D_MODEL), lambda g, be: (be[g0 + g], 0, 0)),
        pl.BlockSpec((1, 1, D_MODEL), lambda g, be: (be[g0 + g], 0, 0)),
        pl.BlockSpec((1, 1, BROW), lambda g, be: (g, 0, 0)),
    ]
    args = [block_expert, xs_half, W, b.reshape(NUM_EXPERTS, 1, D_MODEL),
            wgt_half.reshape(hg, 1, BROW)]
    body = _gmm_body
    aliases = {}
    if ys_prev is not None:
        in_specs = in_specs + [pl.BlockSpec(memory_space=pl.ANY)]
        args = args + [ys_prev]
        body = _gmm_body2
        aliases = {5: 0}
    return pl.pallas_call(
        body,
        grid_spec=pltpu.PrefetchScalarGridSpec(
            num_scalar_prefetch=1,
            grid=(hg,),
            in_specs=in_specs,
            out_specs=pl.BlockSpec((BROW, D_MODEL), lambda g, be: (g0 + g, 0)),
        ),
        out_shape=jax.ShapeDtypeStruct((L_PAD, D_MODEL), jnp.float32),
        input_output_aliases=aliases,
        compiler_params=pltpu.CompilerParams(
            dimension_semantics=("arbitrary",)),
    )(*args)


COMB_CT = 8
COMB_TW = N_TOKENS // SC_WORKERS


def _combine_body(ys_hbm, pos_hbm, out_hbm, pidx_v, g0, g1, ob0, ob1,
                  gsem0, gsem1, wsem0, wsem1):
    nch = COMB_TW // COMB_CT
    wid = lax.axis_index("s") * SC_CORES + lax.axis_index("c")
    base_t = wid * COMB_TW
    pltpu.sync_copy(pos_hbm.at[pl.ds(TOP_K * base_t, TOP_K * COMB_TW)], pidx_v)
    gbufs = (g0, g1)
    obufs = (ob0, ob1)
    gsems = (gsem0, gsem1)
    wsems = (wsem0, wsem1)

    def gather_desc(c, b):
        return pltpu.make_async_copy(
            ys_hbm.at[pidx_v.at[pl.ds(c * TOP_K * COMB_CT, TOP_K * COMB_CT)]],
            gbufs[b], gsems[b])

    def write_desc(c, b):
        return pltpu.make_async_copy(
            obufs[b], out_hbm.at[pl.ds(base_t + c * COMB_CT, COMB_CT)], wsems[b])

    def compute(b):
        gb = gbufs[b]
        obuf = obufs[b]
        for t in range(COMB_CT):
            @plsc.parallel_loop(0, D_MODEL // 16, unroll=8)
            def _(j):
                sl = pl.ds(j * 16, 16)
                obuf[t, sl] = gb[2 * t, sl] + gb[2 * t + 1, sl]

    gather_desc(0, 0).start()

    @pl.loop(0, nch, step=2)
    def _(q):
        gather_desc(q, 0).wait()
        gather_desc(q + 1, 1).start()

        @pl.when(q >= 2)
        def _():
            write_desc(q - 2, 0).wait()
        compute(0)
        write_desc(q, 0).start()

        @pl.when(q + 2 < nch)
        def _():
            gather_desc(q + 2, 0).start()
        gather_desc(q + 1, 1).wait()

        @pl.when(q >= 1)
        def _():
            write_desc(q - 1, 1).wait()
        compute(1)
        write_desc(q + 1, 1).start()

    write_desc(nch - 2, 0).wait()
    write_desc(nch - 1, 1).wait()


def _sc_combine(ys, pos):
    mesh = plsc.VectorSubcoreMesh(core_axis_name="c", subcore_axis_name="s")
    f = pl.kernel(
        _combine_body,
        jax.ShapeDtypeStruct((N_TOKENS, D_MODEL), jnp.float32),
        mesh=mesh,
        scratch_types=[
            pltpu.VMEM((TOP_K * COMB_TW,), jnp.int32),
            pltpu.VMEM((TOP_K * COMB_CT, D_MODEL), jnp.float32),
            pltpu.VMEM((TOP_K * COMB_CT, D_MODEL), jnp.float32),
            pltpu.VMEM((COMB_CT, D_MODEL), jnp.float32),
            pltpu.VMEM((COMB_CT, D_MODEL), jnp.float32),
            pltpu.SemaphoreType.DMA,
            pltpu.SemaphoreType.DMA,
            pltpu.SemaphoreType.DMA,
            pltpu.SemaphoreType.DMA,
        ],
    )
    return f(ys, pos)


@jax.jit
def kernel(x, Wg, W, b):
    w2, sel2, rank2, cnt = _gate(x, Wg)
    tok_pad, wgt_pad, block_expert, pos = _routing_metadata(w2, sel2, rank2, cnt)
    xs1 = _sc_dispatch(x, tok_pad, 0, N_HALF)
    xs2 = _sc_dispatch(x, tok_pad, N_HALF, N_HALF)
    ys1 = _grouped_matmul_half(xs1, W, b, wgt_pad[:N_HALF], block_expert,
                               0, None)
    ys = _grouped_matmul_half(xs2, W, b, wgt_pad[N_HALF:], block_expert,
                              G_BLOCKS // 2, ys1)
    return _sc_combine(ys, pos)


# speedup vs baseline: 1.0542x; 1.0051x over previous
import functools

import jax
import jax.numpy as jnp
from jax import lax
from jax.experimental import pallas as pl
from jax.experimental.pallas import tpu as pltpu
from jax.experimental.pallas import tpu_sc as plsc

NUM_EXPERTS = 8
TOP_K = 2
D_MODEL = 2048
N_TOKENS = 8192
N_ASSIGN = N_TOKENS * TOP_K

BT = 512
BROW = 256
L_PAD = N_ASSIGN + NUM_EXPERTS * BROW
G_BLOCKS = L_PAD // BROW

SC_CORES = 2
SC_SUBCORES = 16
SC_WORKERS = SC_CORES * SC_SUBCORES


def _gate_body(x_ref, wg_ref, w_out_ref, sel_out_ref):
    x = x_ref[...]
    logits = jax.lax.dot_general(
        x, wg_ref[...], (((1,), (1,)), ((), ())),
        preferred_element_type=jnp.float32)
    neg_inf = jnp.float32(-jnp.inf)
    m1 = jnp.full((BT,), neg_inf)
    a1 = jnp.zeros((BT,), jnp.float32)
    for j in range(NUM_EXPERTS):
        lj = logits[:, j]
        better = lj > m1
        m1 = jnp.where(better, lj, m1)
        a1 = jnp.where(better, jnp.float32(j), a1)
    m2 = jnp.full((BT,), neg_inf)
    a2 = jnp.zeros((BT,), jnp.float32)
    for j in range(NUM_EXPERTS):
        lj = logits[:, j]
        valid = jnp.float32(j) != a1
        better = (lj > m2) & valid
        m2 = jnp.where(better, lj, m2)
        a2 = jnp.where(better, jnp.float32(j), a2)
    e2 = jnp.exp(m2 - m1)
    w1 = 1.0 / (1.0 + e2)
    w2 = 1.0 - w1
    w_out_ref[...] = jnp.stack([w1, w2], axis=1)
    sel_out_ref[...] = jnp.stack([a1, a2], axis=1).astype(jnp.int32)


def _gate(x, Wg):
    return pl.pallas_call(
        _gate_body,
        grid=(N_TOKENS // BT,),
        in_specs=[
            pl.BlockSpec((BT, D_MODEL), lambda t: (t, 0)),
            pl.BlockSpec((NUM_EXPERTS, D_MODEL), lambda t: (0, 0)),
        ],
        out_specs=[
            pl.BlockSpec((BT, TOP_K), lambda t: (t, 0)),
            pl.BlockSpec((BT, TOP_K), lambda t: (t, 0)),
        ],
        out_shape=[
            jax.ShapeDtypeStruct((N_TOKENS, TOP_K), jnp.float32),
            jax.ShapeDtypeStruct((N_TOKENS, TOP_K), jnp.int32),
        ],
    )(x, Wg)


def _routing_metadata(w2, sel2):
    e_flat = sel2.reshape(-1)
    w_flat = w2.reshape(-1)
    t_flat = jnp.arange(N_ASSIGN, dtype=jnp.int32) // TOP_K
    onehot = (e_flat[:, None] == jnp.arange(NUM_EXPERTS)[None, :]).astype(jnp.int32)
    ranks_all = jnp.cumsum(onehot, axis=0) - 1
    rank = jnp.take_along_axis(ranks_all, e_flat[:, None], axis=1)[:, 0]
    counts = ranks_all[-1] + 1
    caps = ((counts + BROW - 1) // BROW) * BROW
    P = jnp.concatenate([jnp.zeros((1,), jnp.int32),
                         jnp.cumsum(caps).astype(jnp.int32)])
    pos = P[e_flat] + rank
    tok_pad = jnp.zeros((L_PAD,), jnp.int32).at[pos].set(t_flat)
    wgt_pad = jnp.zeros((L_PAD,), jnp.float32).at[pos].set(w_flat)
    row0 = jnp.arange(G_BLOCKS, dtype=jnp.int32) * BROW
    block_expert = jnp.clip(
        jnp.searchsorted(P, row0, side="right").astype(jnp.int32) - 1,
        0, NUM_EXPERTS - 1)
    return tok_pad, wgt_pad, block_expert, pos


DISP_CH = 16
DISP_NBUF = 3
N_HALF = L_PAD // 2


def _make_dispatch_body(row0, nrows):
    rw = nrows // SC_WORKERS
    nch = rw // DISP_CH

    def body(x_hbm, tok_hbm, xs_hbm, idx_v, *rest):
        bufs = rest[:DISP_NBUF]
        gsems = rest[DISP_NBUF:2 * DISP_NBUF]
        wsems = rest[2 * DISP_NBUF:3 * DISP_NBUF]
        wid = lax.axis_index("s") * SC_CORES + lax.axis_index("c")
        base = wid * rw
        pltpu.sync_copy(tok_hbm.at[pl.ds(row0 + base, rw)], idx_v)

        def gather_desc(c, b):
            return pltpu.make_async_copy(
                x_hbm.at[idx_v.at[pl.ds(c * DISP_CH, DISP_CH)]],
                bufs[b], gsems[b])

        def write_desc(c, b):
            return pltpu.make_async_copy(
                bufs[b], xs_hbm.at[pl.ds(base + c * DISP_CH, DISP_CH)],
                wsems[b])

        for r in range(DISP_NBUF):
            gather_desc(r, r).start()

        @pl.loop(0, nch, step=DISP_NBUF)
        def _(q):
            for r in range(DISP_NBUF):
                gather_desc(q + r, r).wait()
                write_desc(q + r, r).start()
            for r in range(DISP_NBUF):
                @pl.when(q + r + DISP_NBUF < nch)
                def _():
                    write_desc(q + r, r).wait()
                    gather_desc(q + r + DISP_NBUF, r).start()

        for r in range(DISP_NBUF):
            write_desc(nch - DISP_NBUF + r, r).wait()

    return body


def _sc_dispatch(x, tok_pad, row0, nrows):
    mesh = plsc.VectorSubcoreMesh(core_axis_name="c", subcore_axis_name="s")
    rw = nrows // SC_WORKERS
    f = pl.kernel(
        _make_dispatch_body(row0, nrows),
        jax.ShapeDtypeStruct((nrows, D_MODEL), jnp.float32),
        mesh=mesh,
        scratch_types=(
            [pltpu.VMEM((rw,), jnp.int32)]
            + [pltpu.VMEM((DISP_CH, D_MODEL), jnp.float32)] * DISP_NBUF
            + [pltpu.SemaphoreType.DMA] * (2 * DISP_NBUF)
        ),
    )
    return f(x, tok_pad)


def _gmm_body(be_ref, xs_ref, w_ref, b_ref, wgt_ref, o_ref):
    del be_ref
    xs = xs_ref[...]
    y = jax.lax.dot_general(
        xs, w_ref[0], (((1,), (1,)), ((), ())),
        preferred_element_type=jnp.float32) + b_ref[0]
    o_ref[...] = y * wgt_ref[0, 0][:, None]


def _gmm_body2(be_ref, xs_ref, w_ref, b_ref, wgt_ref, prev_ref, o_ref):
    del prev_ref
    _gmm_body(be_ref, xs_ref, w_ref, b_ref, wgt_ref, o_ref)


def _grouped_matmul_half(xs_half, W, b, wgt_half, block_expert, g0, ys_prev):
    hg = G_BLOCKS // 2
    in_specs = [
        pl.BlockSpec((BROW, D_MODEL), lambda g, be: (g, 0)),
        pl.BlockSpec((1, D_MODEL, D_MODEL), lambda g, be: (be[g0 + g], 0, 0)),
        pl.BlockSpec((1, 1, D_MODEL), lambda g, be: (be[g0 + g], 0, 0)),
        pl.BlockSpec((1, 1, BROW), lambda g, be: (g, 0, 0)),
    ]
    args = [block_expert, xs_half, W, b.reshape(NUM_EXPERTS, 1, D_MODEL),
            wgt_half.reshape(hg, 1, BROW)]
    body = _gmm_body
    aliases = {}
    if ys_prev is not None:
        in_specs = in_specs + [pl.BlockSpec(memory_space=pl.ANY)]
        args = args + [ys_prev]
        body = _gmm_body2
        aliases = {5: 0}
    return pl.pallas_call(
        body,
        grid_spec=pltpu.PrefetchScalarGridSpec(
            num_scalar_prefetch=1,
            grid=(hg,),
            in_specs=in_specs,
            out_specs=pl.BlockSpec((BROW, D_MODEL), lambda g, be: (g0 + g, 0)),
        ),
        out_shape=jax.ShapeDtypeStruct((L_PAD, D_MODEL), jnp.float32),
        input_output_aliases=aliases,
        compiler_params=pltpu.CompilerParams(
            dimension_semantics=("arbitrary",)),
    )(*args)


COMB_CT = 8
COMB_TW = N_TOKENS // SC_WORKERS


def _combine_body(ys_hbm, pos_hbm, out_hbm, pidx_v, g0, g1, ob0, ob1,
                  gsem0, gsem1, wsem0, wsem1):
    nch = COMB_TW // COMB_CT
    wid = lax.axis_index("s") * SC_CORES + lax.axis_index("c")
    base_t = wid * COMB_TW
    pltpu.sync_copy(pos_hbm.at[pl.ds(TOP_K * base_t, TOP_K * COMB_TW)], pidx_v)
    gbufs = (g0, g1)
    obufs = (ob0, ob1)
    gsems = (gsem0, gsem1)
    wsems = (wsem0, wsem1)

    def gather_desc(c, b):
        return pltpu.make_async_copy(
            ys_hbm.at[pidx_v.at[pl.ds(c * TOP_K * COMB_CT, TOP_K * COMB_CT)]],
            gbufs[b], gsems[b])

    def write_desc(c, b):
        return pltpu.make_async_copy(
            obufs[b], out_hbm.at[pl.ds(base_t + c * COMB_CT, COMB_CT)], wsems[b])

    def compute(b):
        gb = gbufs[b]
        obuf = obufs[b]
        for t in range(COMB_CT):
            @plsc.parallel_loop(0, D_MODEL // 16, unroll=8)
            def _(j):
                sl = pl.ds(j * 16, 16)
                obuf[t, sl] = gb[2 * t, sl] + gb[2 * t + 1, sl]

    gather_desc(0, 0).start()

    @pl.loop(0, nch, step=2)
    def _(q):
        gather_desc(q, 0).wait()
        gather_desc(q + 1, 1).start()

        @pl.when(q >= 2)
        def _():
            write_desc(q - 2, 0).wait()
        compute(0)
        write_desc(q, 0).start()

        @pl.when(q + 2 < nch)
        def _():
            gather_desc(q + 2, 0).start()
        gather_desc(q + 1, 1).wait()

        @pl.when(q >= 1)
        def _():
            write_desc(q - 1, 1).wait()
        compute(1)
        write_desc(q + 1, 1).start()

    write_desc(nch - 2, 0).wait()
    write_desc(nch - 1, 1).wait()


def _sc_combine(ys, pos):
    mesh = plsc.VectorSubcoreMesh(core_axis_name="c", subcore_axis_name="s")
    f = pl.kernel(
        _combine_body,
        jax.ShapeDtypeStruct((N_TOKENS, D_MODEL), jnp.float32),
        mesh=mesh,
        scratch_types=[
            pltpu.VMEM((TOP_K * COMB_TW,), jnp.int32),
            pltpu.VMEM((TOP_K * COMB_CT, D_MODEL), jnp.float32),
            pltpu.VMEM((TOP_K * COMB_CT, D_MODEL), jnp.float32),
            pltpu.VMEM((COMB_CT, D_MODEL), jnp.float32),
            pltpu.VMEM((COMB_CT, D_MODEL), jnp.float32),
            pltpu.SemaphoreType.DMA,
            pltpu.SemaphoreType.DMA,
            pltpu.SemaphoreType.DMA,
            pltpu.SemaphoreType.DMA,
        ],
    )
    return f(ys, pos)


@jax.jit
def kernel(x, Wg, W, b):
    w2, sel2 = _gate(x, Wg)
    tok_pad, wgt_pad, block_expert, pos = _routing_metadata(w2, sel2)
    xs1 = _sc_dispatch(x, tok_pad, 0, N_HALF)
    xs2 = _sc_dispatch(x, tok_pad, N_HALF, N_HALF)
    ys1 = _grouped_matmul_half(xs1, W, b, wgt_pad[:N_HALF], block_expert,
                               0, None)
    ys = _grouped_matmul_half(xs2, W, b, wgt_pad[N_HALF:], block_expert,
                              G_BLOCKS // 2, ys1)
    return _sc_combine(ys, pos)
